# baseline (speedup 1.0000x reference)
import functools

import jax
import jax.numpy as jnp
from jax import lax
from jax.experimental import pallas as pl
from jax.experimental.pallas import tpu as pltpu
from jax.experimental.pallas import tpu_sc as plsc

NC = 2
NS = 16


@functools.lru_cache(maxsize=None)
def _sc_aggregate(N, D, E):
    NW = NC * NS
    e_per_w = E // NW
    K = 80
    n_chunks = e_per_w // K
    zrows = (N // (NS * 8)) * 8
    zrem = N - zrows * NS
    assert e_per_w * NW == E and n_chunks * K == e_per_w
    assert zrem <= K and zrem % 8 == 0 and zrows % 8 == 0
    assert D % 16 == 0

    mesh = plsc.VectorSubcoreMesh(core_axis_name="c", subcore_axis_name="s")

    @functools.partial(
        pl.kernel,
        out_type=jax.ShapeDtypeStruct((NC, N, D), jnp.float32),
        mesh=mesh,
        scratch_types=[
            pltpu.VMEM((e_per_w,), jnp.int32),
            pltpu.VMEM((e_per_w,), jnp.int32),
            pltpu.VMEM((2, K), jnp.float32),
            pltpu.VMEM((2, K), jnp.int32),
            pltpu.VMEM((2, K), jnp.int32),
            pltpu.VMEM((2, K, D), jnp.float32),
            pltpu.VMEM_SHARED((N, D), jnp.float32),
            pltpu.SemaphoreType.DMA((2,)),
            pltpu.SemaphoreType.DMA((2,)),
            pltpu.SemaphoreType.DMA((2,)),
        ],
    )
    def agg(x_hbm, src_hbm, dst_hbm, w_hbm, out_hbm,
            src_all, dst_all, w_sm, src_sm, dst_sm, rows_v,
            acc_sh, gsem, wsem, ssem):
        c = lax.axis_index("c")
        s = lax.axis_index("s")
        wid = c * NS + s

        zvec = jnp.zeros((16,), jnp.float32)

        def zrow(r, carry):
            for cb in range(D // 16):
                rows_v[0, r, pl.ds(cb * 16, 16)] = zvec
            return carry

        lax.fori_loop(0, K, zrow, 0)
        n_zfull, ztail = zrows // K, zrows % K
        for z in range(n_zfull):
            pltpu.sync_copy(rows_v.at[0], acc_sh.at[pl.ds(s * zrows + z * K, K)])
        if ztail:
            pltpu.sync_copy(rows_v.at[0, pl.ds(0, ztail)],
                            acc_sh.at[pl.ds(s * zrows + n_zfull * K, ztail)])
        if zrem:
            @pl.when(s == 0)
            def _():
                pltpu.sync_copy(rows_v.at[0, pl.ds(0, zrem)],
                                acc_sh.at[pl.ds(NS * zrows, zrem)])
        plsc.subcore_barrier()

        base = wid * e_per_w
        pltpu.sync_copy(src_hbm.at[pl.ds(base, e_per_w)], src_all)
        pltpu.sync_copy(dst_hbm.at[pl.ds(base, e_per_w)], dst_all)

        def start_gather(i, b):
            for t in range(K // 16):
                src_sm[b, pl.ds(t * 16, 16)] = src_all[pl.ds(i * K + t * 16, 16)]
            pltpu.async_copy(w_hbm.at[pl.ds(base + i * K, K)], w_sm.at[b],
                             wsem.at[b])
            return pltpu.async_copy(x_hbm.at[src_sm.at[b]], rows_v.at[b], gsem.at[b])

        start_gather(0, 0)

        def chunk(i, carry):
            b = lax.rem(i, 2)
            nb = 1 - b
            pltpu.make_async_copy(x_hbm.at[src_sm.at[b]], rows_v.at[b],
                                  gsem.at[b]).wait()
            pltpu.make_async_copy(w_hbm.at[pl.ds(base + i * K, K)], w_sm.at[b],
                                  wsem.at[b]).wait()
            for t in range(K // 16):
                dst_sm[b, pl.ds(t * 16, 16)] = dst_all[pl.ds(i * K + t * 16, 16)]

            def edge16(t, carry2):
                wv = w_sm[b, pl.ds(t * 16, 16)]
                for l in range(16):
                    wj = wv[l]
                    j = t * 16 + l
                    for cb in range(D // 16):
                        rows_v[b, j, pl.ds(cb * 16, 16)] = (
                            rows_v[b, j, pl.ds(cb * 16, 16)] * wj
                        )
                return carry2

            lax.fori_loop(0, K // 16, edge16, 0)
            pltpu.async_copy(rows_v.at[b], acc_sh.at[dst_sm.at[b]],
                             ssem.at[b], add=True)

            @pl.when(i + 1 < n_chunks)
            def _():
                @pl.when(i >= 1)
                def _():
                    pltpu.make_async_copy(rows_v.at[nb],
                                          acc_sh.at[dst_sm.at[nb]],
                                          ssem.at[nb]).wait()
                start_gather(i + 1, nb)

            return carry

        lax.fori_loop(0, n_chunks, chunk, 0)
        for last in (n_chunks - 2, n_chunks - 1):
            lb = last % 2
            pltpu.make_async_copy(rows_v.at[lb], acc_sh.at[dst_sm.at[lb]],
                                  ssem.at[lb]).wait()
        plsc.subcore_barrier()

        @pl.when(s == 0)
        def _():
            pltpu.sync_copy(acc_sh, out_hbm.at[c])

    return agg


@functools.lru_cache(maxsize=None)
def _tc_finish(N, D):
    BLK = 1000
    assert N % BLK == 0

    def body(p_ref, w_ref, b_ref, o_ref):
        acc = p_ref[0] + p_ref[1]
        o_ref[...] = (
            jnp.dot(acc, w_ref[...], preferred_element_type=jnp.float32)
            + b_ref[...]
        )

    return pl.pallas_call(
        body,
        grid=(N // BLK,),
        in_specs=[
            pl.BlockSpec((NC, BLK, D), lambda i: (0, i, 0)),
            pl.BlockSpec((D, D), lambda i: (0, 0)),
            pl.BlockSpec((1, D), lambda i: (0, 0)),
        ],
        out_specs=pl.BlockSpec((BLK, D), lambda i: (i, 0)),
        out_shape=jax.ShapeDtypeStruct((N, D), jnp.float32),
    )


def kernel(x, edge_index, edge_weight, W, b):
    N, D = x.shape
    E = edge_weight.shape[0]
    partials = _sc_aggregate(N, D, E)(
        x, edge_index[0], edge_index[1], edge_weight)
    return _tc_finish(N, D)(partials, W, b.reshape(1, D))


# speedup vs baseline: 2.8285x; 2.8285x over previous
import functools

import jax
import jax.numpy as jnp
from jax import lax
from jax.experimental import pallas as pl
from jax.experimental.pallas import tpu as pltpu
from jax.experimental.pallas import tpu_sc as plsc

NC = 2
NS = 16


@functools.lru_cache(maxsize=None)
def _sc_aggregate(N, D, E):
    NW = NC * NS
    e_per_w = E // NW
    K = 80
    n_chunks = e_per_w // K
    zrows = (N // (NS * 8)) * 8
    zrem = N - zrows * NS
    assert e_per_w * NW == E and n_chunks * K == e_per_w
    assert n_chunks % 2 == 1
    assert zrem <= K and zrem % 8 == 0 and zrows % 8 == 0
    assert D % 16 == 0

    mesh = plsc.VectorSubcoreMesh(core_axis_name="c", subcore_axis_name="s")

    @functools.partial(
        pl.kernel,
        out_type=jax.ShapeDtypeStruct((NC, N, D), jnp.float32),
        mesh=mesh,
        scratch_types=[
            pltpu.VMEM((e_per_w,), jnp.int32),
            pltpu.VMEM((e_per_w,), jnp.int32),
            pltpu.VMEM((2, K), jnp.float32),
            pltpu.VMEM((2, K), jnp.int32),
            pltpu.VMEM((2, K), jnp.int32),
            pltpu.VMEM((2, K, D), jnp.float32),
            pltpu.VMEM_SHARED((N, D), jnp.float32),
            pltpu.SemaphoreType.DMA((2,)),
            pltpu.SemaphoreType.DMA((2,)),
            pltpu.SemaphoreType.DMA((2,)),
        ],
    )
    def agg(x_hbm, src_hbm, dst_hbm, w_hbm, out_hbm,
            src_all, dst_all, w_sm, src_sm, dst_sm, rows_v,
            acc_sh, gsem, wsem, ssem):
        c = lax.axis_index("c")
        s = lax.axis_index("s")
        wid = c * NS + s

        zvec = jnp.zeros((16,), jnp.float32)

        def zrow(r, carry):
            for cb in range(D // 16):
                rows_v[0, r, pl.ds(cb * 16, 16)] = zvec
            return carry

        lax.fori_loop(0, K, zrow, 0)
        n_zfull, ztail = zrows // K, zrows % K
        for z in range(n_zfull):
            pltpu.sync_copy(rows_v.at[0], acc_sh.at[pl.ds(s * zrows + z * K, K)])
        if ztail:
            pltpu.sync_copy(rows_v.at[0, pl.ds(0, ztail)],
                            acc_sh.at[pl.ds(s * zrows + n_zfull * K, ztail)])
        if zrem:
            @pl.when(s == 0)
            def _():
                pltpu.sync_copy(rows_v.at[0, pl.ds(0, zrem)],
                                acc_sh.at[pl.ds(NS * zrows, zrem)])
        plsc.subcore_barrier()

        base = wid * e_per_w
        pltpu.sync_copy(src_hbm.at[pl.ds(base, e_per_w)], src_all)
        pltpu.sync_copy(dst_hbm.at[pl.ds(base, e_per_w)], dst_all)

        def start_gather(i, b):
            for t in range(K // 16):
                src_sm[b, pl.ds(t * 16, 16)] = src_all[pl.ds(i * K + t * 16, 16)]
            pltpu.async_copy(w_hbm.at[pl.ds(base + i * K, K)], w_sm.at[b],
                             wsem.at[b])
            return pltpu.async_copy(x_hbm.at[src_sm.at[b]], rows_v.at[b], gsem.at[b])

        def process(i, b, prefetch_next):
            pltpu.make_async_copy(x_hbm.at[src_sm.at[b]], rows_v.at[b],
                                  gsem.at[b]).wait()
            pltpu.make_async_copy(w_hbm.at[pl.ds(base + i * K, K)], w_sm.at[b],
                                  wsem.at[b]).wait()
            if prefetch_next:
                start_gather(i + 1, 1 - b)
            for t in range(K // 16):
                dst_sm[b, pl.ds(t * 16, 16)] = dst_all[pl.ds(i * K + t * 16, 16)]

            def edge16(t, carry2):
                wv = w_sm[b, pl.ds(t * 16, 16)]
                for l in range(16):
                    wj = wv[l]
                    j = t * 16 + l
                    for cb in range(D // 16):
                        rows_v[b, j, pl.ds(cb * 16, 16)] = (
                            rows_v[b, j, pl.ds(cb * 16, 16)] * wj
                        )
                return carry2

            lax.fori_loop(0, K // 16, edge16, 0)
            pltpu.sync_copy(rows_v.at[b], acc_sh.at[dst_sm.at[b]], add=True)

        start_gather(0, 0)

        def pair(t, carry):
            i = t * 2
            process(i, 0, prefetch_next=True)
            process(i + 1, 1, prefetch_next=True)
            return carry

        lax.fori_loop(0, n_chunks // 2, pair, 0)
        if n_chunks % 2:
            process(n_chunks - 1, 0, prefetch_next=False)
        plsc.subcore_barrier()

        @pl.when(s == 0)
        def _():
            pltpu.sync_copy(acc_sh, out_hbm.at[c])

    return agg


@functools.lru_cache(maxsize=None)
def _tc_finish(N, D):
    BLK = 1000
    assert N % BLK == 0

    def body(p_ref, w_ref, b_ref, o_ref):
        acc = p_ref[0] + p_ref[1]
        o_ref[...] = (
            jnp.dot(acc, w_ref[...], preferred_element_type=jnp.float32)
            + b_ref[...]
        )

    return pl.pallas_call(
        body,
        grid=(N // BLK,),
        in_specs=[
            pl.BlockSpec((NC, BLK, D), lambda i: (0, i, 0)),
            pl.BlockSpec((D, D), lambda i: (0, 0)),
            pl.BlockSpec((1, D), lambda i: (0, 0)),
        ],
        out_specs=pl.BlockSpec((BLK, D), lambda i: (i, 0)),
        out_shape=jax.ShapeDtypeStruct((N, D), jnp.float32),
    )


def kernel(x, edge_index, edge_weight, W, b):
    N, D = x.shape
    E = edge_weight.shape[0]
    partials = _sc_aggregate(N, D, E)(
        x, edge_index[0], edge_index[1], edge_weight)
    return _tc_finish(N, D)(partials, W, b.reshape(1, D))
